# baseline (speedup 1.0000x reference)
import jax
import jax.numpy as jnp
from jax.experimental import pallas as pl
from jax.experimental.pallas import tpu as pltpu


_BM = 2000


def _mm_body(q_ref, mk_ref, o_ref):
    mk = mk_ref[...].astype(jnp.bfloat16)
    qb = q_ref[...].astype(jnp.bfloat16)
    o_ref[...] = jnp.dot(mk, qb, preferred_element_type=jnp.float32)


def kernel(q, memory_key):
    m, k = memory_key.shape
    b = q.shape[1]
    grid = (m // _BM,)
    return pl.pallas_call(
        _mm_body,
        grid=grid,
        in_specs=[
            pl.BlockSpec((k, b), lambda i: (0, 0)),
            pl.BlockSpec((_BM, k), lambda i: (i, 0)),
        ],
        out_specs=pl.BlockSpec((_BM, b), lambda i: (i, 0)),
        out_shape=jax.ShapeDtypeStruct((m, b), jnp.float32),
        compiler_params=pltpu.CompilerParams(
            dimension_semantics=("parallel",),
        ),
    )(q, memory_key)


# speedup vs baseline: 1.0239x; 1.0239x over previous
import jax
import jax.numpy as jnp
from jax.experimental import pallas as pl
from jax.experimental.pallas import tpu as pltpu


_BM = 4000


def _mm_body(q_ref, mk_ref, o_ref):
    mk = mk_ref[...].astype(jnp.bfloat16)
    qb = q_ref[...].astype(jnp.bfloat16)
    o_ref[...] = jnp.dot(mk, qb, preferred_element_type=jnp.float32)


def kernel(q, memory_key):
    m, k = memory_key.shape
    b = q.shape[1]
    grid = (m // _BM,)
    return pl.pallas_call(
        _mm_body,
        grid=grid,
        in_specs=[
            pl.BlockSpec((k, b), lambda i: (0, 0)),
            pl.BlockSpec((_BM, k), lambda i: (i, 0)),
        ],
        out_specs=pl.BlockSpec((_BM, b), lambda i: (i, 0)),
        out_shape=jax.ShapeDtypeStruct((m, b), jnp.float32),
        compiler_params=pltpu.CompilerParams(
            dimension_semantics=("parallel",),
        ),
    )(q, memory_key)


# speedup vs baseline: 1.0286x; 1.0046x over previous
import jax
import jax.numpy as jnp
from jax.experimental import pallas as pl
from jax.experimental.pallas import tpu as pltpu


_BM = 5000


def _mm_body(q_ref, mk_ref, o_ref):
    mk = mk_ref[...].astype(jnp.bfloat16)
    qb = q_ref[...].astype(jnp.bfloat16)
    o_ref[...] = jnp.dot(mk, qb, preferred_element_type=jnp.float32)


def kernel(q, memory_key):
    m, k = memory_key.shape
    b = q.shape[1]
    grid = (m // _BM,)
    return pl.pallas_call(
        _mm_body,
        grid=grid,
        in_specs=[
            pl.BlockSpec((k, b), lambda i: (0, 0)),
            pl.BlockSpec((_BM, k), lambda i: (i, 0)),
        ],
        out_specs=pl.BlockSpec((_BM, b), lambda i: (i, 0)),
        out_shape=jax.ShapeDtypeStruct((m, b), jnp.float32),
        compiler_params=pltpu.CompilerParams(
            dimension_semantics=("parallel",),
        ),
    )(q, memory_key)
